# baseline (speedup 1.0000x reference)
import functools

import jax
import jax.numpy as jnp
from jax import lax
from jax.experimental import pallas as pl
from jax.experimental.pallas import tpu as pltpu
from jax.experimental.pallas import tpu_sc as plsc

N_NODES = 10000
NOF = 128
H1 = 128
H2 = 64

NC = 2
NS = 16
NW = NC * NS
LANES = 128

N_PAD = 10240
ROWS_N = N_PAD // NS

E = 320000
EW = 64
E_TROWS = 160
E_ROWS = NW * E_TROWS
E_PAD = E_ROWS * EW

ED = 640000
ED_TROWS = 160
ED_ROWS = NW * ED_TROWS
ED_PAD = ED_ROWS * LANES

_MESH = plsc.VectorSubcoreMesh(
    core_axis_name="c", subcore_axis_name="s", num_cores=NC, num_subcores=NS)

_f32 = jnp.float32
_i32 = jnp.int32


def _deg_body(dst_hbm, out_hbm, idx_v, ones_v, buf_v, acc_sh):
    c = lax.axis_index("c")
    s = lax.axis_index("s")
    w = s * NC + c
    for i in range(EW // 16):
        ones_v[pl.ds(i * 16, 16)] = jnp.ones((16,), _f32)
    for i in range(ROWS_N // 16):
        buf_v[pl.ds(i * 16, 16)] = jnp.zeros((16,), _f32)
    pltpu.sync_copy(buf_v, acc_sh.at[pl.ds(s * ROWS_N, ROWS_N)])
    plsc.subcore_barrier()
    pltpu.sync_copy(dst_hbm.at[pl.ds(w * E_TROWS, E_TROWS)], idx_v)

    @pl.loop(0, E_TROWS)
    def _(j):
        pltpu.sync_copy(ones_v, acc_sh.at[idx_v.at[j]], add=True)

    plsc.subcore_barrier()
    pltpu.sync_copy(acc_sh.at[pl.ds(s * ROWS_N, ROWS_N)], buf_v)
    pltpu.sync_copy(buf_v, out_hbm.at[c, pl.ds(s * ROWS_N, ROWS_N)])


_deg_call = pl.kernel(
    _deg_body,
    out_type=jax.ShapeDtypeStruct((NC, N_PAD), _f32),
    mesh=_MESH,
    compiler_params=pltpu.CompilerParams(use_tc_tiling_on_sc=False, needs_layout_passes=False),
    scratch_types=[
        pltpu.VMEM((E_TROWS, EW), _i32),
        pltpu.VMEM((EW,), _f32),
        pltpu.VMEM((ROWS_N,), _f32),
        pltpu.VMEM_SHARED((N_PAD,), _f32),
    ],
)

_ZR = 32


def _scatter_body(D, width, trows, xs_hbm, src_hbm, dst_hbm, out_hbm,
                  sidx, didx, rows, zer, acc_sh, sem):
    c = lax.axis_index("c")
    s = lax.axis_index("s")
    w = s * NC + c
    for r in range(_ZR):
        for k in range(D // 16):
            zer[r, pl.ds(k * 16, 16)] = jnp.zeros((16,), _f32)
    for t in range(ROWS_N // _ZR):
        pltpu.sync_copy(zer, acc_sh.at[pl.ds(s * ROWS_N + t * _ZR, _ZR)])
    plsc.subcore_barrier()
    pltpu.sync_copy(src_hbm.at[pl.ds(w * trows, trows)], sidx)
    pltpu.sync_copy(dst_hbm.at[pl.ds(w * trows, trows)], didx)

    rows_a, rows_b = rows
    gsem_a, gsem_b, ssem_a, ssem_b = sem
    pltpu.async_copy(xs_hbm.at[sidx.at[0]], rows_a, gsem_a)

    @pl.loop(0, trows // 2)
    def _(g):
        j0 = 2 * g

        @pl.when(g > 0)
        def _():
            pltpu.make_async_copy(rows_b, acc_sh.at[didx.at[j0 - 1]], ssem_b).wait()

        pltpu.async_copy(xs_hbm.at[sidx.at[j0 + 1]], rows_b, gsem_b)
        pltpu.make_async_copy(xs_hbm.at[sidx.at[j0]], rows_a, gsem_a).wait()
        pltpu.async_copy(rows_a, acc_sh.at[didx.at[j0]], ssem_a, add=True)
        pltpu.make_async_copy(xs_hbm.at[sidx.at[j0 + 1]], rows_b, gsem_b).wait()
        pltpu.async_copy(rows_b, acc_sh.at[didx.at[j0 + 1]], ssem_b, add=True)
        pltpu.make_async_copy(rows_a, acc_sh.at[didx.at[j0]], ssem_a).wait()

        @pl.when(g < trows // 2 - 1)
        def _():
            pltpu.async_copy(xs_hbm.at[sidx.at[j0 + 2]], rows_a, gsem_a)

    pltpu.make_async_copy(rows_b, acc_sh.at[didx.at[trows - 1]], ssem_b).wait()

    plsc.subcore_barrier()
    for t in range(ROWS_N // width):
        pltpu.sync_copy(acc_sh.at[pl.ds(s * ROWS_N + t * width, width)], rows_a)
        pltpu.sync_copy(rows_a, out_hbm.at[c, pl.ds(s * ROWS_N + t * width, width)])


def _make_scatter(D, width, trows):
    return pl.kernel(
        functools.partial(_scatter_body, D, width, trows),
        out_type=jax.ShapeDtypeStruct((NC, N_PAD, D), _f32),
        mesh=_MESH,
        compiler_params=pltpu.CompilerParams(use_tc_tiling_on_sc=False, needs_layout_passes=False),
        scratch_types=[
            pltpu.VMEM((trows, width), _i32),
            pltpu.VMEM((trows, width), _i32),
            (pltpu.VMEM((width, D), _f32), pltpu.VMEM((width, D), _f32)),
            pltpu.VMEM((_ZR, D), _f32),
            pltpu.VMEM_SHARED((N_PAD, D), _f32),
            (pltpu.SemaphoreType.DMA, pltpu.SemaphoreType.DMA,
             pltpu.SemaphoreType.DMA, pltpu.SemaphoreType.DMA),
        ],
    )


_scatter128 = _make_scatter(H1, EW, E_TROWS)
_scatter64 = _make_scatter(H2, LANES, E_PAD // (NW * LANES))


def _dgather_body(z_hbm, src_hbm, dst_hbm, zs_out, zd_out,
                  sidx, didx, srows, drows, gsem, wsem):
    c = lax.axis_index("c")
    s = lax.axis_index("s")
    w = s * NC + c
    pltpu.sync_copy(src_hbm.at[pl.ds(w * ED_TROWS, ED_TROWS)], sidx)
    pltpu.sync_copy(dst_hbm.at[pl.ds(w * ED_TROWS, ED_TROWS)], didx)

    def fire(j, b):
        pltpu.async_copy(z_hbm.at[sidx.at[j]], srows[b], gsem[2 * b])
        pltpu.async_copy(z_hbm.at[didx.at[j]], drows[b], gsem[2 * b + 1])

    def wait_g(j, b):
        pltpu.make_async_copy(z_hbm.at[sidx.at[j]], srows[b], gsem[2 * b]).wait()
        pltpu.make_async_copy(z_hbm.at[didx.at[j]], drows[b], gsem[2 * b + 1]).wait()

    def fire_w(j, b):
        pltpu.async_copy(srows[b], zs_out.at[w * ED_TROWS + j], wsem[2 * b])
        pltpu.async_copy(drows[b], zd_out.at[w * ED_TROWS + j], wsem[2 * b + 1])

    def wait_w(j, b):
        pltpu.make_async_copy(srows[b], zs_out.at[w * ED_TROWS + j], wsem[2 * b]).wait()
        pltpu.make_async_copy(drows[b], zd_out.at[w * ED_TROWS + j], wsem[2 * b + 1]).wait()

    fire(0, 0)
    fire(1, 1)

    @pl.loop(0, ED_TROWS // 4)
    def _(g):
        j0 = 4 * g
        fire(j0 + 2, 2)
        fire(j0 + 3, 3)
        wait_g(j0, 0)
        fire_w(j0, 0)
        wait_g(j0 + 1, 1)
        fire_w(j0 + 1, 1)

        @pl.when(g < ED_TROWS // 4 - 1)
        def _():
            wait_w(j0, 0)
            wait_w(j0 + 1, 1)
            fire(j0 + 4, 0)
            fire(j0 + 5, 1)

        wait_g(j0 + 2, 2)
        fire_w(j0 + 2, 2)
        wait_g(j0 + 3, 3)
        fire_w(j0 + 3, 3)
        wait_w(j0 + 2, 2)
        wait_w(j0 + 3, 3)

    wait_w(ED_TROWS - 4, 0)
    wait_w(ED_TROWS - 3, 1)


_bf16 = jnp.bfloat16

_dgather_call = pl.kernel(
    _dgather_body,
    out_type=[
        jax.ShapeDtypeStruct((ED_ROWS, LANES, H2), _bf16),
        jax.ShapeDtypeStruct((ED_ROWS, LANES, H2), _bf16),
    ],
    mesh=_MESH,
    compiler_params=pltpu.CompilerParams(use_tc_tiling_on_sc=False, needs_layout_passes=False),
    scratch_types=[
        pltpu.VMEM((ED_TROWS, LANES), _i32),
        pltpu.VMEM((ED_TROWS, LANES), _i32),
        tuple(pltpu.VMEM((LANES, H2), _bf16) for _ in range(4)),
        tuple(pltpu.VMEM((LANES, H2), _bf16) for _ in range(4)),
        tuple(pltpu.SemaphoreType.DMA for _ in range(8)),
        tuple(pltpu.SemaphoreType.DMA for _ in range(8)),
    ],
)


def _dot_kernel(a_ref, b_ref, o_ref):
    o_ref[...] = jnp.sum(a_ref[...].astype(_f32) * b_ref[...].astype(_f32), axis=-1)


_DOT_BLK = 8192


def _edge_dots(zs, zd):
    return pl.pallas_call(
        _dot_kernel,
        grid=(ED_PAD // _DOT_BLK,),
        in_specs=[
            pl.BlockSpec((_DOT_BLK, H2), lambda i: (i, 0)),
            pl.BlockSpec((_DOT_BLK, H2), lambda i: (i, 0)),
        ],
        out_specs=pl.BlockSpec((_DOT_BLK,), lambda i: (i,)),
        out_shape=jax.ShapeDtypeStruct((ED_PAD,), _f32),
    )(zs, zd)


_B = 1024


def _tc1_kernel(x_ref, w_ref, degp_ref, xs_ref, dis_ref):
    xw = jnp.dot(x_ref[...], w_ref[...], preferred_element_type=_f32)
    deg = degp_ref[0] + degp_ref[1] + 1.0
    dis = lax.rsqrt(deg)
    xs_ref[...] = xw * dis[:, None]
    dis_ref[...] = dis


def _tc1(x_pad, W1, degp):
    return pl.pallas_call(
        _tc1_kernel,
        grid=(N_PAD // _B,),
        in_specs=[
            pl.BlockSpec((_B, NOF), lambda i: (i, 0)),
            pl.BlockSpec((NOF, H1), lambda i: (0, 0)),
            pl.BlockSpec((NC, _B), lambda i: (0, i)),
        ],
        out_specs=[
            pl.BlockSpec((_B, H1), lambda i: (i, 0)),
            pl.BlockSpec((_B,), lambda i: (i,)),
        ],
        out_shape=[
            jax.ShapeDtypeStruct((N_PAD, H1), _f32),
            jax.ShapeDtypeStruct((N_PAD,), _f32),
        ],
    )(x_pad, W1, degp)


def _tc2_kernel(acc_ref, xs1_ref, dis_ref, b1_ref, w2_ref, xs2_ref):
    dis = dis_ref[...]
    m = acc_ref[0] + acc_ref[1] + xs1_ref[...]
    h = jnp.maximum(dis[:, None] * m + b1_ref[...], 0.0)
    xs2_ref[...] = jnp.dot(h, w2_ref[...], preferred_element_type=_f32) * dis[:, None]


def _tc2(acc1, xs1, dis, b1, W2):
    return pl.pallas_call(
        _tc2_kernel,
        grid=(N_PAD // _B,),
        in_specs=[
            pl.BlockSpec((NC, _B, H1), lambda i: (0, i, 0)),
            pl.BlockSpec((_B, H1), lambda i: (i, 0)),
            pl.BlockSpec((_B,), lambda i: (i,)),
            pl.BlockSpec((H1,), lambda i: (0,)),
            pl.BlockSpec((H1, H2), lambda i: (0, 0)),
        ],
        out_specs=pl.BlockSpec((_B, H2), lambda i: (i, 0)),
        out_shape=jax.ShapeDtypeStruct((N_PAD, H2), _f32),
    )(acc1, xs1, dis, b1, W2)


def _tc3_kernel(acc_ref, xs2_ref, dis_ref, b2_ref, z_ref):
    z = (dis_ref[...][:, None]
         * (acc_ref[0] + acc_ref[1] + xs2_ref[...]) + b2_ref[...])
    z_ref[...] = z.astype(jnp.bfloat16)


def _tc3(acc2, xs2, dis, b2):
    return pl.pallas_call(
        _tc3_kernel,
        grid=(N_PAD // _B,),
        in_specs=[
            pl.BlockSpec((NC, _B, H2), lambda i: (0, i, 0)),
            pl.BlockSpec((_B, H2), lambda i: (i, 0)),
            pl.BlockSpec((_B,), lambda i: (i,)),
            pl.BlockSpec((H2,), lambda i: (0,)),
        ],
        out_specs=pl.BlockSpec((_B, H2), lambda i: (i, 0)),
        out_shape=jax.ShapeDtypeStruct((N_PAD, H2), jnp.bfloat16),
    )(acc2, xs2, dis, b2)


def _pad_indices(n):
    return N_NODES + (jnp.arange(n, dtype=_i32) % (N_PAD - N_NODES))


def kernel(x, train_pos_edge_index, neg_edge_index, W1, b1, W2, b2):
    src = train_pos_edge_index[0]
    dst = train_pos_edge_index[1]
    pad_e = _pad_indices(E_PAD - E)
    src_flat = jnp.concatenate([src, pad_e])
    dst_flat = jnp.concatenate([dst, pad_e])
    src2d = src_flat.reshape(E_ROWS, EW)
    dst2d = dst_flat.reshape(E_ROWS, EW)
    src2dw = src_flat.reshape(-1, LANES)
    dst2dw = dst_flat.reshape(-1, LANES)
    x_pad = jnp.pad(x, ((0, N_PAD - N_NODES), (0, 0)))

    degp = _deg_call(dst2d)
    xs1, dis = _tc1(x_pad, W1, degp)
    acc1 = _scatter128(xs1, src2d, dst2d)
    xs2 = _tc2(acc1, xs1, dis, b1, W2)
    acc2 = _scatter64(xs2, src2dw, dst2dw)
    z = _tc3(acc2, xs2, dis, b2)

    pad_d = _pad_indices(ED_PAD - ED)
    dsrc = jnp.concatenate(
        [train_pos_edge_index[0], neg_edge_index[0], pad_d]).reshape(ED_ROWS, LANES)
    ddst = jnp.concatenate(
        [train_pos_edge_index[1], neg_edge_index[1], pad_d]).reshape(ED_ROWS, LANES)
    zs_g, zd_g = _dgather_call(z, dsrc, ddst)
    logits = _edge_dots(zs_g.reshape(ED_PAD, H2), zd_g.reshape(ED_PAD, H2))
    return logits[:ED]


# speedup vs baseline: 2.0931x; 2.0931x over previous
import functools

import jax
import jax.numpy as jnp
from jax import lax
from jax.experimental import pallas as pl
from jax.experimental.pallas import tpu as pltpu
from jax.experimental.pallas import tpu_sc as plsc

N_NODES = 10000
NOF = 128
H1 = 128
H2 = 64

NC = 2
NS = 16
NW = NC * NS
LANES = 128

N_PAD = 10240
ROWS_N = N_PAD // NS

E = 320000
EW = 64
E_TROWS = 160
E_ROWS = NW * E_TROWS
E_PAD = E_ROWS * EW

ED = 640000
ED_TROWS = 158
ED_ROWS = NW * ED_TROWS
ED_PAD = ED_ROWS * LANES

_MESH = plsc.VectorSubcoreMesh(
    core_axis_name="c", subcore_axis_name="s", num_cores=NC, num_subcores=NS)

_f32 = jnp.float32
_i32 = jnp.int32


def _deg_body(dst_hbm, out_hbm, idx_v, ones_v, buf_v, acc_sh):
    c = lax.axis_index("c")
    s = lax.axis_index("s")
    w = s * NC + c
    for i in range(EW // 16):
        ones_v[pl.ds(i * 16, 16)] = jnp.ones((16,), _f32)
    for i in range(ROWS_N // 16):
        buf_v[pl.ds(i * 16, 16)] = jnp.zeros((16,), _f32)
    pltpu.sync_copy(buf_v, acc_sh.at[pl.ds(s * ROWS_N, ROWS_N)])
    plsc.subcore_barrier()
    pltpu.sync_copy(dst_hbm.at[pl.ds(w * E_TROWS, E_TROWS)], idx_v)

    @pl.loop(0, E_TROWS)
    def _(j):
        pltpu.sync_copy(ones_v, acc_sh.at[idx_v.at[j]], add=True)

    plsc.subcore_barrier()
    pltpu.sync_copy(acc_sh.at[pl.ds(s * ROWS_N, ROWS_N)], buf_v)
    pltpu.sync_copy(buf_v, out_hbm.at[c, pl.ds(s * ROWS_N, ROWS_N)])


_deg_call = pl.kernel(
    _deg_body,
    out_type=jax.ShapeDtypeStruct((NC, N_PAD), _f32),
    mesh=_MESH,
    compiler_params=pltpu.CompilerParams(use_tc_tiling_on_sc=False, needs_layout_passes=False),
    scratch_types=[
        pltpu.VMEM((E_TROWS, EW), _i32),
        pltpu.VMEM((EW,), _f32),
        pltpu.VMEM((ROWS_N,), _f32),
        pltpu.VMEM_SHARED((N_PAD,), _f32),
    ],
)

_ZR = 32


def _scatter_body(D, width, trows, xs_hbm, src_hbm, dst_hbm, out_hbm,
                  sidx, didx, rows, zer, acc_sh, sem):
    c = lax.axis_index("c")
    s = lax.axis_index("s")
    w = s * NC + c
    for r in range(_ZR):
        for k in range(D // 16):
            zer[r, pl.ds(k * 16, 16)] = jnp.zeros((16,), _f32)
    for t in range(ROWS_N // _ZR):
        pltpu.sync_copy(zer, acc_sh.at[pl.ds(s * ROWS_N + t * _ZR, _ZR)])
    plsc.subcore_barrier()
    pltpu.sync_copy(src_hbm.at[pl.ds(w * trows, trows)], sidx)
    pltpu.sync_copy(dst_hbm.at[pl.ds(w * trows, trows)], didx)

    rows_a, rows_b = rows
    gsem_a, gsem_b, ssem_a, ssem_b = sem
    pltpu.async_copy(xs_hbm.at[sidx.at[0]], rows_a, gsem_a)

    @pl.loop(0, trows // 2)
    def _(g):
        j0 = 2 * g

        @pl.when(g > 0)
        def _():
            pltpu.make_async_copy(rows_b, acc_sh.at[didx.at[j0 - 1]], ssem_b).wait()

        pltpu.async_copy(xs_hbm.at[sidx.at[j0 + 1]], rows_b, gsem_b)
        pltpu.make_async_copy(xs_hbm.at[sidx.at[j0]], rows_a, gsem_a).wait()
        pltpu.async_copy(rows_a, acc_sh.at[didx.at[j0]], ssem_a, add=True)
        pltpu.make_async_copy(xs_hbm.at[sidx.at[j0 + 1]], rows_b, gsem_b).wait()
        pltpu.async_copy(rows_b, acc_sh.at[didx.at[j0 + 1]], ssem_b, add=True)
        pltpu.make_async_copy(rows_a, acc_sh.at[didx.at[j0]], ssem_a).wait()

        @pl.when(g < trows // 2 - 1)
        def _():
            pltpu.async_copy(xs_hbm.at[sidx.at[j0 + 2]], rows_a, gsem_a)

    pltpu.make_async_copy(rows_b, acc_sh.at[didx.at[trows - 1]], ssem_b).wait()

    plsc.subcore_barrier()
    for t in range(ROWS_N // width):
        pltpu.sync_copy(acc_sh.at[pl.ds(s * ROWS_N + t * width, width)], rows_a)
        pltpu.sync_copy(rows_a, out_hbm.at[c, pl.ds(s * ROWS_N + t * width, width)])


def _make_scatter(D, width, trows):
    return pl.kernel(
        functools.partial(_scatter_body, D, width, trows),
        out_type=jax.ShapeDtypeStruct((NC, N_PAD, D), _f32),
        mesh=_MESH,
        compiler_params=pltpu.CompilerParams(use_tc_tiling_on_sc=False, needs_layout_passes=False),
        scratch_types=[
            pltpu.VMEM((trows, width), _i32),
            pltpu.VMEM((trows, width), _i32),
            (pltpu.VMEM((width, D), _f32), pltpu.VMEM((width, D), _f32)),
            pltpu.VMEM((_ZR, D), _f32),
            pltpu.VMEM_SHARED((N_PAD, D), _f32),
            (pltpu.SemaphoreType.DMA, pltpu.SemaphoreType.DMA,
             pltpu.SemaphoreType.DMA, pltpu.SemaphoreType.DMA),
        ],
    )


_scatter128 = _make_scatter(H1, EW, E_TROWS)
_scatter64 = _make_scatter(H2, LANES, E_PAD // (NW * LANES))


_bf16 = jnp.bfloat16


def _decode_body(z_hbm, src_hbm, dst_hbm, out_hbm,
                 sidx, didx, srows, drows, colbuf, outv, sem):
    c = lax.axis_index("c")
    s = lax.axis_index("s")
    w = s * NC + c
    sra, srb = srows
    dra, drb = drows
    sem_sa, sem_da, sem_sb, sem_db = sem
    pltpu.sync_copy(src_hbm.at[pl.ds(w * ED_TROWS, ED_TROWS)], sidx)
    pltpu.sync_copy(dst_hbm.at[pl.ds(w * ED_TROWS, ED_TROWS)], didx)
    iota = lax.iota(_i32, 16)

    def fire(j, sr, dr, ss, sd):
        pltpu.async_copy(z_hbm.at[sidx.at[j]], sr, ss)
        pltpu.async_copy(z_hbm.at[didx.at[j]], dr, sd)

    def wait(j, sr, dr, ss, sd):
        pltpu.make_async_copy(z_hbm.at[sidx.at[j]], sr, ss).wait()
        pltpu.make_async_copy(z_hbm.at[didx.at[j]], dr, sd).wait()

    def compute_row(sr, dr, j):
        for e in range(LANES):
            acc = None
            for k in range(H2 // 32):
                p = sr[e, pl.ds(k * 32, 32)] * dr[e, pl.ds(k * 32, 32)]
                p0, p1 = plsc.unpack(p, format=plsc.PackFormat.INTERLEAVED)
                q = p0 + p1
                acc = q if acc is None else acc + q
            plsc.store_scatter(colbuf, [iota, jnp.full((16,), e, _i32)], acc)
        for g in range(LANES // 16):
            t0 = colbuf[0, pl.ds(g * 16, 16)]
            for l in range(1, 16):
                t0 = t0 + colbuf[l, pl.ds(g * 16, 16)]
            outv[pl.ds(g * 16, 16)] = t0
        pltpu.sync_copy(outv, out_hbm.at[w * ED_TROWS + j])

    fire(0, sra, dra, sem_sa, sem_da)

    @pl.loop(0, ED_TROWS // 2)
    def _(g):
        j0 = 2 * g
        wait(j0, sra, dra, sem_sa, sem_da)
        fire(j0 + 1, srb, drb, sem_sb, sem_db)
        compute_row(sra, dra, j0)
        wait(j0 + 1, srb, drb, sem_sb, sem_db)

        @pl.when(g < ED_TROWS // 2 - 1)
        def _():
            fire(j0 + 2, sra, dra, sem_sa, sem_da)

        compute_row(srb, drb, j0 + 1)


_decode_call = pl.kernel(
    _decode_body,
    out_type=jax.ShapeDtypeStruct((ED_ROWS, LANES), _f32),
    mesh=_MESH,
    compiler_params=pltpu.CompilerParams(use_tc_tiling_on_sc=False, needs_layout_passes=False),
    scratch_types=[
        pltpu.VMEM((ED_TROWS, LANES), _i32),
        pltpu.VMEM((ED_TROWS, LANES), _i32),
        (pltpu.VMEM((LANES, H2), _bf16), pltpu.VMEM((LANES, H2), _bf16)),
        (pltpu.VMEM((LANES, H2), _bf16), pltpu.VMEM((LANES, H2), _bf16)),
        pltpu.VMEM((16, 129), _f32),
        pltpu.VMEM((LANES,), _f32),
        (pltpu.SemaphoreType.DMA, pltpu.SemaphoreType.DMA,
         pltpu.SemaphoreType.DMA, pltpu.SemaphoreType.DMA),
    ],
)


_B = 1024


def _tc1_kernel(x_ref, w_ref, degp_ref, xs_ref, dis_ref):
    xw = jnp.dot(x_ref[...], w_ref[...], preferred_element_type=_f32)
    deg = degp_ref[0] + degp_ref[1] + 1.0
    dis = lax.rsqrt(deg)
    xs_ref[...] = xw * dis[:, None]
    dis_ref[...] = dis


def _tc1(x_pad, W1, degp):
    return pl.pallas_call(
        _tc1_kernel,
        grid=(N_PAD // _B,),
        in_specs=[
            pl.BlockSpec((_B, NOF), lambda i: (i, 0)),
            pl.BlockSpec((NOF, H1), lambda i: (0, 0)),
            pl.BlockSpec((NC, _B), lambda i: (0, i)),
        ],
        out_specs=[
            pl.BlockSpec((_B, H1), lambda i: (i, 0)),
            pl.BlockSpec((_B,), lambda i: (i,)),
        ],
        out_shape=[
            jax.ShapeDtypeStruct((N_PAD, H1), _f32),
            jax.ShapeDtypeStruct((N_PAD,), _f32),
        ],
    )(x_pad, W1, degp)


def _tc2_kernel(acc_ref, xs1_ref, dis_ref, b1_ref, w2_ref, xs2_ref):
    dis = dis_ref[...]
    m = acc_ref[0] + acc_ref[1] + xs1_ref[...]
    h = jnp.maximum(dis[:, None] * m + b1_ref[...], 0.0)
    xs2_ref[...] = jnp.dot(h, w2_ref[...], preferred_element_type=_f32) * dis[:, None]


def _tc2(acc1, xs1, dis, b1, W2):
    return pl.pallas_call(
        _tc2_kernel,
        grid=(N_PAD // _B,),
        in_specs=[
            pl.BlockSpec((NC, _B, H1), lambda i: (0, i, 0)),
            pl.BlockSpec((_B, H1), lambda i: (i, 0)),
            pl.BlockSpec((_B,), lambda i: (i,)),
            pl.BlockSpec((H1,), lambda i: (0,)),
            pl.BlockSpec((H1, H2), lambda i: (0, 0)),
        ],
        out_specs=pl.BlockSpec((_B, H2), lambda i: (i, 0)),
        out_shape=jax.ShapeDtypeStruct((N_PAD, H2), _f32),
    )(acc1, xs1, dis, b1, W2)


def _tc3_kernel(acc_ref, xs2_ref, dis_ref, b2_ref, z_ref):
    z = (dis_ref[...][:, None]
         * (acc_ref[0] + acc_ref[1] + xs2_ref[...]) + b2_ref[...])
    z_ref[...] = z.astype(jnp.bfloat16)


def _tc3(acc2, xs2, dis, b2):
    return pl.pallas_call(
        _tc3_kernel,
        grid=(N_PAD // _B,),
        in_specs=[
            pl.BlockSpec((NC, _B, H2), lambda i: (0, i, 0)),
            pl.BlockSpec((_B, H2), lambda i: (i, 0)),
            pl.BlockSpec((_B,), lambda i: (i,)),
            pl.BlockSpec((H2,), lambda i: (0,)),
        ],
        out_specs=pl.BlockSpec((_B, H2), lambda i: (i, 0)),
        out_shape=jax.ShapeDtypeStruct((N_PAD, H2), jnp.bfloat16),
    )(acc2, xs2, dis, b2)


def _pad_indices(n):
    return N_NODES + (jnp.arange(n, dtype=_i32) % (N_PAD - N_NODES))


def kernel(x, train_pos_edge_index, neg_edge_index, W1, b1, W2, b2):
    src = train_pos_edge_index[0]
    dst = train_pos_edge_index[1]
    pad_e = _pad_indices(E_PAD - E)
    src_flat = jnp.concatenate([src, pad_e])
    dst_flat = jnp.concatenate([dst, pad_e])
    src2d = src_flat.reshape(E_ROWS, EW)
    dst2d = dst_flat.reshape(E_ROWS, EW)
    src2dw = src_flat.reshape(-1, LANES)
    dst2dw = dst_flat.reshape(-1, LANES)
    x_pad = jnp.pad(x, ((0, N_PAD - N_NODES), (0, 0)))

    degp = _deg_call(dst2d)
    xs1, dis = _tc1(x_pad, W1, degp)
    acc1 = _scatter128(xs1, src2d, dst2d)
    xs2 = _tc2(acc1, xs1, dis, b1, W2)
    acc2 = _scatter64(xs2, src2dw, dst2dw)
    z = _tc3(acc2, xs2, dis, b2)

    pad_d = _pad_indices(ED_PAD - ED)
    dsrc = jnp.concatenate(
        [train_pos_edge_index[0], neg_edge_index[0], pad_d]).reshape(ED_ROWS, LANES)
    ddst = jnp.concatenate(
        [train_pos_edge_index[1], neg_edge_index[1], pad_d]).reshape(ED_ROWS, LANES)
    logits2d = _decode_call(z, dsrc, ddst)
    return logits2d.reshape(-1)[:ED]
